# baseline (speedup 1.0000x reference)
import functools

import jax
import jax.numpy as jnp
from jax import lax
from jax.experimental import pallas as pl
from jax.experimental.pallas import tpu as pltpu
from jax.experimental.pallas import tpu_sc as plsc

_info = plsc.get_sparse_core_info()
_NC, _NS = _info.num_cores, _info.num_subcores
_NW = _NC * _NS

_G = 32


def _make_lookup(R, T, V, D):
    assert R % _NW == 0
    r_per_w = R // _NW
    assert r_per_w % _G == 0
    ngroups = r_per_w // _G
    nfull = T // 16
    ntail = T - 16 * nfull
    mesh = plsc.VectorSubcoreMesh(core_axis_name="c", subcore_axis_name="s")

    @functools.partial(
        pl.kernel,
        mesh=mesh,
        out_type=jax.ShapeDtypeStruct((R, T, D), jnp.float32),
        scratch_types=[
            [pltpu.VMEM((_G, T), jnp.float32) for _ in range(2)],
            [pltpu.VMEM((_G, T), jnp.int32) for _ in range(2)],
            [pltpu.VMEM((_G, T, D), jnp.float32) for _ in range(2)],
            [pltpu.SemaphoreType.DMA for _ in range(2)],
            [pltpu.SemaphoreType.DMA for _ in range(2)],
            [pltpu.SemaphoreType.DMA for _ in range(2)],
        ],
        compiler_params=pltpu.CompilerParams(
            use_tc_tiling_on_sc=False, needs_layout_passes=False
        ),
    )
    def k(idx_hbm, table_hbm, out_hbm, fbufs, qbufs, rbufs, isems, gsems, wsems):
        wid = lax.axis_index("s") * _NC + lax.axis_index("c")
        rbase = pl.multiple_of(wid * r_per_w, r_per_w)
        lane = lax.iota(jnp.int32, 16)

        icopies = [None, None]
        gcopies = [[None] * _G, [None] * _G]
        wcopies = [None, None]

        def start_idx(g):
            b = g % 2
            icopies[b] = pltpu.async_copy(
                idx_hbm.at[pl.ds(rbase + g * _G, _G)], fbufs[b], isems[b]
            )

        def convert(b):
            for j in range(_G):
                for s in range(nfull):
                    v = fbufs[b][j, pl.ds(16 * s, 16)]
                    qbufs[b][j, pl.ds(16 * s, 16)] = v.astype(jnp.int32)
            if ntail:
                assert ntail == 2
                per = 16 // ntail
                jv = lane >> 1
                tv = (16 * nfull) + (lane & 1)
                for j0 in range(0, _G, per):
                    vals = plsc.load_gather(fbufs[b], [jv + j0, tv])
                    plsc.store_scatter(
                        qbufs[b], [jv + j0, tv], vals.astype(jnp.int32)
                    )

        start_idx(0)
        for g in range(ngroups):
            b = g % 2
            if g >= 2:
                wcopies[b].wait()
            icopies[b].wait()
            if g + 1 < ngroups:
                start_idx(g + 1)
            convert(b)
            for j in range(_G):
                gcopies[b][j] = pltpu.async_copy(
                    table_hbm.at[qbufs[b].at[j]], rbufs[b].at[j], gsems[b]
                )
            for j in range(_G):
                gcopies[b][j].wait()
            wcopies[b] = pltpu.async_copy(
                rbufs[b], out_hbm.at[pl.ds(rbase + g * _G, _G)], wsems[b]
            )
        wcopies[(ngroups - 2) % 2].wait()
        wcopies[(ngroups - 1) % 2].wait()

    return k


def kernel(token_ids, weight):
    V, D = weight.shape
    R, T = token_ids.shape
    q = R // 8
    f = _make_lookup(q, T, V, D)
    outs = [
        f(token_ids[i * q:(i + 1) * q].astype(jnp.float32), weight)
        for i in range(8)
    ]
    return jnp.concatenate(outs, axis=0)


# speedup vs baseline: 1.0083x; 1.0083x over previous
import functools

import jax
import jax.numpy as jnp
from jax import lax
from jax.experimental import pallas as pl
from jax.experimental.pallas import tpu as pltpu
from jax.experimental.pallas import tpu_sc as plsc

_info = plsc.get_sparse_core_info()
_NC, _NS = _info.num_cores, _info.num_subcores
_NW = _NC * _NS

_G = 32


def _make_lookup(R, T, V, D):
    assert R % _NW == 0
    r_per_w = R // _NW
    assert r_per_w % _G == 0
    ngroups = r_per_w // _G
    nfull = T // 16
    ntail = T - 16 * nfull
    mesh = plsc.VectorSubcoreMesh(core_axis_name="c", subcore_axis_name="s")

    @functools.partial(
        pl.kernel,
        mesh=mesh,
        out_type=jax.ShapeDtypeStruct((R, T, D), jnp.float32),
        scratch_types=[
            [pltpu.VMEM((_G, T), jnp.float32) for _ in range(2)],
            [pltpu.VMEM((_G, T), jnp.int32) for _ in range(2)],
            [pltpu.VMEM((_G, T, D), jnp.float32) for _ in range(2)],
            [pltpu.SemaphoreType.DMA for _ in range(2)],
            [pltpu.SemaphoreType.DMA for _ in range(2)],
            [pltpu.SemaphoreType.DMA for _ in range(2)],
        ],
        compiler_params=pltpu.CompilerParams(
            use_tc_tiling_on_sc=False, needs_layout_passes=False
        ),
    )
    def k(idx_hbm, table_hbm, out_hbm, fbufs, qbufs, rbufs, isems, gsems, wsems):
        wid = lax.axis_index("s") * _NC + lax.axis_index("c")
        rbase = pl.multiple_of(wid * r_per_w, r_per_w)
        lane = lax.iota(jnp.int32, 16)

        icopies = [None, None]
        gcopies = [[None] * _G, [None] * _G]
        wcopies = [None, None]

        def start_idx(g):
            b = g % 2
            icopies[b] = pltpu.async_copy(
                idx_hbm.at[pl.ds(rbase + g * _G, _G)], fbufs[b], isems[b]
            )

        def convert(b):
            for j in range(_G):
                for s in range(nfull):
                    v = fbufs[b][j, pl.ds(16 * s, 16)]
                    qbufs[b][j, pl.ds(16 * s, 16)] = v.astype(jnp.int32)
            if ntail:
                assert ntail == 2
                per = 16 // ntail
                jv = lane >> 1
                tv = (16 * nfull) + (lane & 1)
                for j0 in range(0, _G, per):
                    vals = plsc.load_gather(fbufs[b], [jv + j0, tv])
                    plsc.store_scatter(
                        qbufs[b], [jv + j0, tv], vals.astype(jnp.int32)
                    )

        start_idx(0)
        for g in range(ngroups):
            b = g % 2
            if g >= 2:
                wcopies[b].wait()
            icopies[b].wait()
            if g + 1 < ngroups:
                start_idx(g + 1)
            convert(b)
            for j in range(_G):
                gcopies[b][j] = pltpu.async_copy(
                    table_hbm.at[qbufs[b].at[j]], rbufs[b].at[j], gsems[b]
                )
            for j in range(_G):
                gcopies[b][j].wait()
            wcopies[b] = pltpu.async_copy(
                rbufs[b], out_hbm.at[pl.ds(rbase + g * _G, _G)], wsems[b]
            )
        wcopies[(ngroups - 2) % 2].wait()
        wcopies[(ngroups - 1) % 2].wait()

    return k


def kernel(token_ids, weight):
    V, D = weight.shape
    R, T = token_ids.shape
    q = R // 4
    f = _make_lookup(q, T, V, D)
    outs = [
        f(token_ids[i * q:(i + 1) * q].astype(jnp.float32), weight)
        for i in range(4)
    ]
    return jnp.concatenate(outs, axis=0)
